# baseline (speedup 1.0000x reference)
import functools

import jax
import jax.numpy as jnp
from jax import lax
from jax.experimental import pallas as pl
from jax.experimental.pallas import tpu as pltpu
from jax.experimental.pallas import tpu_sc as plsc

N = 10000
E = 160000
D = 32
G = 200
GP = 256
EP = 163840
NP = 10016
BE = 2048
NW = 32
EW = EP // NW
CH = 1024
NCH = EW // CH
ROWS_T = NP // 16


def _h0_body(xp_ref, w_ref, b_ref, o_ref):
    o_ref[...] = jnp.maximum(
        jnp.dot(xp_ref[...], w_ref[...], preferred_element_type=jnp.float32)
        + b_ref[...], 0.0)


def _msg_body(ea_ref, hs_ref, en1_ref, b1_ref, wcat_ref, sel_ref, o_ref):
    z = jnp.maximum(
        jnp.dot(ea_ref[...], en1_ref[...], preferred_element_type=jnp.float32)
        + b1_ref[...], 0.0)
    oc = jnp.where(lax.broadcasted_iota(jnp.int32, (BE, 8), 1) == 0, 1.0, 0.0)
    z1 = jnp.concatenate([z, oc], axis=1).astype(jnp.bfloat16)
    zb = jnp.dot(z1, sel_ref[...],
                 preferred_element_type=jnp.float32)
    hsb = hs_ref[...].astype(jnp.bfloat16)
    acc = jnp.dot(hsb, wcat_ref[:, 0:128],
                  preferred_element_type=jnp.float32) * zb[:, 0:128]
    for g in range(1, 8):
        pg = jnp.dot(hsb, wcat_ref[:, 128 * g:128 * (g + 1)],
                     preferred_element_type=jnp.float32)
        acc = acc + pg * zb[:, 128 * g:128 * (g + 1)]
    msg = jnp.dot(hsb, wcat_ref[:, D * D:D * D + D],
                  preferred_element_type=jnp.float32)
    for r in range(4):
        msg = msg + acc[:, D * r:D * (r + 1)]
    o_ref[...] = msg


def _update_body(p_ref, h_ref, root_ref, cb_ref, wih_ref, whh_ref,
                 bih_ref, bhh_ref, o_ref):
    h = h_ref[...]
    agg = p_ref[0:N, :] + p_ref[NP:NP + N, :]
    m = jnp.maximum(
        agg + jnp.dot(h, root_ref[...], preferred_element_type=jnp.float32)
        + cb_ref[...], 0.0)
    gx = jnp.dot(m, wih_ref[...], preferred_element_type=jnp.float32) + bih_ref[...]
    gh = jnp.dot(h, whh_ref[...], preferred_element_type=jnp.float32) + bhh_ref[...]
    r = jax.nn.sigmoid(gx[:, 0:D] + gh[:, 0:D])
    zz = jax.nn.sigmoid(gx[:, D:2 * D] + gh[:, D:2 * D])
    n = jnp.tanh(gx[:, 2 * D:3 * D] + r * gh[:, 2 * D:3 * D])
    o_ref[...] = (1.0 - zz) * n + zz * h


def _lstm(x, h, c, wih, whh, bih, bhh):
    g = (jnp.dot(x, wih, preferred_element_type=jnp.float32) + bih
         + jnp.dot(h, whh, preferred_element_type=jnp.float32) + bhh)
    i = jax.nn.sigmoid(g[:, 0:D])
    f = jax.nn.sigmoid(g[:, D:2 * D])
    gg = jnp.tanh(g[:, 2 * D:3 * D])
    o = jax.nn.sigmoid(g[:, 3 * D:4 * D])
    c = f * c + i * gg
    return jax.nn.sigmoid(g[:, 3 * D:4 * D]) * jnp.tanh(c), c


def _s2s_body(h_ref, b_ref, s2s_wih_ref, s2s_whh_ref, s2s_bih_ref, s2s_bhh_ref,
              mem_wih_ref, mem_whh_ref, mem_bih_ref, mem_bhh_ref,
              mlp1_ref, mlp1b_ref, mlp2_ref, mlp2b_ref,
              v_ref, hx_ref, cx_ref):
    out = h_ref[...]
    gid = lax.broadcasted_iota(jnp.int32, (1, GP), 1)
    mask = (b_ref[...] == gid)
    mf = mask.astype(jnp.float32)
    qh = jnp.zeros((GP, D), jnp.float32)
    qc = jnp.zeros((GP, D), jnp.float32)
    q_star = jnp.zeros((GP, 2 * D), jnp.float32)
    for _ in range(6):
        qh, qc = _lstm(q_star, qh, qc, s2s_wih_ref[...], s2s_whh_ref[...],
                       s2s_bih_ref[...], s2s_bhh_ref[...])
        qhb = jnp.dot(mf, qh, preferred_element_type=jnp.float32)
        e = jnp.sum(out * qhb, axis=1, keepdims=True)
        emax = jnp.max(jnp.where(mask, e, -1e30), axis=0, keepdims=True)
        emaxb = jnp.sum(mf * emax, axis=1, keepdims=True)
        a = jnp.exp(e - emaxb)
        asum = jnp.sum(mf * a, axis=0, keepdims=True)
        asb = jnp.sum(mf * asum, axis=1, keepdims=True)
        an = a / (asb + 1e-16)
        r = lax.dot_general(mf * an, out, (((0,), (0,)), ((), ())),
                            preferred_element_type=jnp.float32)
        q_star = jnp.concatenate([qh, r], axis=1)
    hx = jnp.zeros((GP, D), jnp.float32)
    cx = jnp.zeros((GP, D), jnp.float32)
    hx, cx = _lstm(q_star, hx, cx, mem_wih_ref[...], mem_whh_ref[...],
                   mem_bih_ref[...], mem_bhh_ref[...])
    hid = jnp.maximum(
        jnp.dot(hx, mlp1_ref[...], preferred_element_type=jnp.float32)
        + mlp1b_ref[...], 0.0)
    v_ref[...] = (jnp.dot(hid, mlp2_ref[...], preferred_element_type=jnp.float32)
                  + mlp2b_ref[...])
    hx_ref[...] = hx
    cx_ref[...] = cx


def _gather_body(h_hbm, src_hbm, out_hbm, idx_v, rows_v, sem):
    wid = lax.axis_index("s") * 2 + lax.axis_index("c")

    def chunk(ch, _):
        ebase = wid * EW + ch * CH
        rbase = wid * (EW // 128) + ch * (CH // 128)
        pltpu.sync_copy(src_hbm.at[pl.ds(rbase, CH // 128)], idx_v)
        descs = [pltpu.async_copy(h_hbm.at[idx_v.at[j]],
                                  rows_v.at[pl.ds(j * 128, 128)], sem)
                 for j in range(CH // 128)]
        for d in descs:
            d.wait()
        pltpu.sync_copy(rows_v, out_hbm.at[pl.ds(ebase, CH)])
        return ()

    lax.fori_loop(0, NCH, chunk, ())


def _scatter_body(msg_hbm, dst_hbm, zrows_hbm, out_hbm, idx_v, msg_v, agg_sh, sem):
    cid = lax.axis_index("c")
    sid = lax.axis_index("s")
    pltpu.sync_copy(zrows_hbm, agg_sh.at[pl.ds(sid * ROWS_T, ROWS_T)])
    plsc.subcore_barrier()

    def chunk(ch, _):
        ebase = cid * (EP // 2) + sid * EW + ch * CH
        rbase = ebase // 128
        pltpu.sync_copy(dst_hbm.at[pl.ds(rbase, CH // 128)], idx_v)
        pltpu.sync_copy(msg_hbm.at[pl.ds(ebase, CH)], msg_v)
        for j in range(CH // 128):
            pltpu.sync_copy(msg_v.at[pl.ds(j * 128, 128)],
                            agg_sh.at[idx_v.at[j]], add=True)
        return ()

    lax.fori_loop(0, NCH, chunk, ())
    plsc.subcore_barrier()
    pltpu.sync_copy(agg_sh.at[pl.ds(sid * ROWS_T, ROWS_T)],
                    out_hbm.at[pl.ds(cid * NP + sid * ROWS_T, ROWS_T)])


@functools.lru_cache(maxsize=1)
def _sc_mesh():
    return plsc.VectorSubcoreMesh(core_axis_name="c", subcore_axis_name="s",
                                  num_cores=2, num_subcores=16)


def _sc_gather(h, src2d):
    f = pl.kernel(
        _gather_body,
        out_type=jax.ShapeDtypeStruct((EP, D), jnp.float32),
        mesh=_sc_mesh(),
        scratch_types=[
            pltpu.VMEM((CH // 128, 128), jnp.int32),
            pltpu.VMEM((CH, D), jnp.float32),
            pltpu.SemaphoreType.DMA,
        ],
        compiler_params=pltpu.CompilerParams(use_tc_tiling_on_sc=False),
    )
    return f(h, src2d)


def _sc_scatter(msg, dst2d, zrows):
    f = pl.kernel(
        _scatter_body,
        out_type=jax.ShapeDtypeStruct((2 * NP, D), jnp.float32),
        mesh=_sc_mesh(),
        scratch_types=[
            pltpu.VMEM((CH // 128, 128), jnp.int32),
            pltpu.VMEM((CH, D), jnp.float32),
            pltpu.VMEM_SHARED((NP, D), jnp.float32),
            pltpu.SemaphoreType.DMA,
        ],
        compiler_params=pltpu.CompilerParams(use_tc_tiling_on_sc=False),
    )
    return f(msg, dst2d, zrows)


def _tc_h0(xp, w0, b0):
    return pl.pallas_call(
        _h0_body,
        out_shape=jax.ShapeDtypeStruct((N, D), jnp.float32),
    )(xp, w0, b0)


def _tc_msg(ea, hs, en1, b1, wcat, sel):
    grid = EP // BE
    return pl.pallas_call(
        _msg_body,
        grid=(grid,),
        in_specs=[
            pl.BlockSpec((BE, 8), lambda i: (i, 0)),
            pl.BlockSpec((BE, D), lambda i: (i, 0)),
            pl.BlockSpec((8, D), lambda i: (0, 0)),
            pl.BlockSpec((1, D), lambda i: (0, 0)),
            pl.BlockSpec((D, D * D + D), lambda i: (0, 0)),
            pl.BlockSpec((40, D * D + D), lambda i: (0, 0)),
        ],
        out_specs=pl.BlockSpec((BE, D), lambda i: (i, 0)),
        out_shape=jax.ShapeDtypeStruct((EP, D), jnp.float32),
    )(ea, hs, en1, b1, wcat, sel)


def _tc_update(p, h, root, cb, wih, whh, bih, bhh):
    return pl.pallas_call(
        _update_body,
        out_shape=jax.ShapeDtypeStruct((N, D), jnp.float32),
    )(p, h, root, cb, wih, whh, bih, bhh)


def _tc_s2s(h, b2d, args):
    return pl.pallas_call(
        _s2s_body,
        out_shape=(
            jax.ShapeDtypeStruct((GP, 1), jnp.float32),
            jax.ShapeDtypeStruct((GP, D), jnp.float32),
            jax.ShapeDtypeStruct((GP, D), jnp.float32),
        ),
    )(h, b2d, *args)


def kernel(x, edge_index, edge_attr, batch, lin0_W, lin0_b, en1_W, en1_b,
           en2_W, en2_b, root, conv_b, gru_Wih, gru_Whh, gru_bih, gru_bhh,
           s2s_Wih, s2s_Whh, s2s_bih, s2s_bhh, mem_Wih, mem_Whh, mem_bih,
           mem_bhh, mlp1_W, mlp1_b, mlp2_W, mlp2_b):
    f32 = jnp.float32
    src = edge_index[0]
    dst = edge_index[1]
    pad = EP - E
    src2d = jnp.concatenate([src, jnp.zeros((pad,), jnp.int32)]).reshape(EP // 128, 128)
    dst2d = jnp.concatenate([dst, jnp.full((pad,), N, jnp.int32)]).reshape(EP // 128, 128)
    ea = jnp.pad(edge_attr, ((0, pad), (0, 4)))
    xp = jnp.pad(x, ((0, 0), (0, 5)))
    w0 = jnp.pad(lin0_W, ((0, 0), (0, 5))).T
    b0 = lin0_b.reshape(1, D)
    en1 = jnp.pad(en1_W, ((0, 0), (0, 4))).T
    b1 = en1_b.reshape(1, D)
    wcat = jnp.concatenate(
        [en2_W.reshape(D, D, D).transpose(0, 2, 1).reshape(D, D * D),
         en2_b.reshape(D, D)], axis=1).astype(jnp.bfloat16)
    kk = jnp.arange(40)[:, None]
    cc = jnp.arange(D * D + D)[None, :]
    sel = (cc // D == kk).astype(jnp.bfloat16)
    wih = gru_Wih.T
    whh = gru_Whh.T
    bih = gru_bih.reshape(1, 3 * D)
    bhh = gru_bhh.reshape(1, 3 * D)
    zrows = jnp.zeros((ROWS_T, D), f32)
    b2d = batch.reshape(N, 1)

    h = _tc_h0(xp, w0, b0)
    for _ in range(6):
        hs = _sc_gather(h, src2d)
        msg = _tc_msg(ea, hs, en1, b1, wcat, sel)
        p = _sc_scatter(msg, dst2d, zrows)
        h = _tc_update(p, h, root, cb := conv_b.reshape(1, D), wih, whh, bih, bhh)

    s2s_args = (s2s_Wih.T, s2s_Whh.T, s2s_bih.reshape(1, 4 * D),
                s2s_bhh.reshape(1, 4 * D), mem_Wih.T, mem_Whh.T,
                mem_bih.reshape(1, 4 * D), mem_bhh.reshape(1, 4 * D),
                mlp1_W.T, mlp1_b.reshape(1, D), mlp2_W.T, mlp2_b.reshape(1, 1))
    v, hx, cx = _tc_s2s(h, b2d, s2s_args)
    return v[:G][None], hx[:G][None], cx[:G][None]


# speedup vs baseline: 1.0307x; 1.0307x over previous
import functools

import jax
import jax.numpy as jnp
from jax import lax
from jax.experimental import pallas as pl
from jax.experimental.pallas import tpu as pltpu
from jax.experimental.pallas import tpu_sc as plsc

N = 10000
E = 160000
D = 32
G = 200
GP = 256
EP = 163840
NP = 10016
BE = 4096
NW = 32
EW = EP // NW
CH = 1024
NCH = EW // CH
ROWS_T = NP // 16


def _h0_body(xp_ref, w_ref, b_ref, o_ref):
    o_ref[...] = jnp.maximum(
        jnp.dot(xp_ref[...], w_ref[...], preferred_element_type=jnp.float32)
        + b_ref[...], 0.0)


def _msg_body(ea_ref, hs_ref, en1_ref, b1_ref, wcat_ref, sel_ref, o_ref):
    z = jnp.maximum(
        jnp.dot(ea_ref[...], en1_ref[...], preferred_element_type=jnp.float32)
        + b1_ref[...], 0.0)
    oc = jnp.where(lax.broadcasted_iota(jnp.int32, (BE, 8), 1) == 0, 1.0, 0.0)
    z1 = jnp.concatenate([z, oc], axis=1).astype(jnp.bfloat16)
    zb = jnp.dot(z1, sel_ref[...],
                 preferred_element_type=jnp.float32)
    p = jnp.dot(hs_ref[...].astype(jnp.bfloat16), wcat_ref[...],
                preferred_element_type=jnp.float32)
    q = p * zb
    s1 = q[:, 0:128]
    for j in range(1, 8):
        s1 = s1 + q[:, 128 * j:128 * (j + 1)]
    msg = q[:, D * D:D * D + D]
    for r in range(4):
        msg = msg + s1[:, D * r:D * (r + 1)]
    o_ref[...] = msg


def _update_body(p_ref, h_ref, root_ref, cb_ref, wih_ref, whh_ref,
                 bih_ref, bhh_ref, o_ref):
    h = h_ref[...]
    agg = p_ref[0:N, :] + p_ref[NP:NP + N, :]
    m = jnp.maximum(
        agg + jnp.dot(h, root_ref[...], preferred_element_type=jnp.float32)
        + cb_ref[...], 0.0)
    gx = jnp.dot(m, wih_ref[...], preferred_element_type=jnp.float32) + bih_ref[...]
    gh = jnp.dot(h, whh_ref[...], preferred_element_type=jnp.float32) + bhh_ref[...]
    r = jax.nn.sigmoid(gx[:, 0:D] + gh[:, 0:D])
    zz = jax.nn.sigmoid(gx[:, D:2 * D] + gh[:, D:2 * D])
    n = jnp.tanh(gx[:, 2 * D:3 * D] + r * gh[:, 2 * D:3 * D])
    o_ref[...] = (1.0 - zz) * n + zz * h


def _lstm(x, h, c, wih, whh, bih, bhh):
    g = (jnp.dot(x, wih, preferred_element_type=jnp.float32) + bih
         + jnp.dot(h, whh, preferred_element_type=jnp.float32) + bhh)
    i = jax.nn.sigmoid(g[:, 0:D])
    f = jax.nn.sigmoid(g[:, D:2 * D])
    gg = jnp.tanh(g[:, 2 * D:3 * D])
    o = jax.nn.sigmoid(g[:, 3 * D:4 * D])
    c = f * c + i * gg
    return jax.nn.sigmoid(g[:, 3 * D:4 * D]) * jnp.tanh(c), c


def _s2s_body(h_ref, b_ref, s2s_wih_ref, s2s_whh_ref, s2s_bih_ref, s2s_bhh_ref,
              mem_wih_ref, mem_whh_ref, mem_bih_ref, mem_bhh_ref,
              mlp1_ref, mlp1b_ref, mlp2_ref, mlp2b_ref,
              v_ref, hx_ref, cx_ref):
    out = h_ref[...]
    gid = lax.broadcasted_iota(jnp.int32, (1, GP), 1)
    mask = (b_ref[...] == gid)
    mf = mask.astype(jnp.float32)
    qh = jnp.zeros((GP, D), jnp.float32)
    qc = jnp.zeros((GP, D), jnp.float32)
    q_star = jnp.zeros((GP, 2 * D), jnp.float32)
    for _ in range(6):
        qh, qc = _lstm(q_star, qh, qc, s2s_wih_ref[...], s2s_whh_ref[...],
                       s2s_bih_ref[...], s2s_bhh_ref[...])
        qhb = jnp.dot(mf, qh, preferred_element_type=jnp.float32)
        e = jnp.sum(out * qhb, axis=1, keepdims=True)
        emax = jnp.max(jnp.where(mask, e, -1e30), axis=0, keepdims=True)
        emaxb = jnp.sum(mf * emax, axis=1, keepdims=True)
        a = jnp.exp(e - emaxb)
        asum = jnp.sum(mf * a, axis=0, keepdims=True)
        asb = jnp.sum(mf * asum, axis=1, keepdims=True)
        an = a / (asb + 1e-16)
        r = lax.dot_general(mf * an, out, (((0,), (0,)), ((), ())),
                            preferred_element_type=jnp.float32)
        q_star = jnp.concatenate([qh, r], axis=1)
    hx = jnp.zeros((GP, D), jnp.float32)
    cx = jnp.zeros((GP, D), jnp.float32)
    hx, cx = _lstm(q_star, hx, cx, mem_wih_ref[...], mem_whh_ref[...],
                   mem_bih_ref[...], mem_bhh_ref[...])
    hid = jnp.maximum(
        jnp.dot(hx, mlp1_ref[...], preferred_element_type=jnp.float32)
        + mlp1b_ref[...], 0.0)
    v_ref[...] = (jnp.dot(hid, mlp2_ref[...], preferred_element_type=jnp.float32)
                  + mlp2b_ref[...])
    hx_ref[...] = hx
    cx_ref[...] = cx


def _gather_body(h_hbm, src_hbm, out_hbm, idx_v, rows_v, sem):
    wid = lax.axis_index("s") * 2 + lax.axis_index("c")

    def chunk(ch, _):
        ebase = wid * EW + ch * CH
        rbase = wid * (EW // 128) + ch * (CH // 128)
        pltpu.sync_copy(src_hbm.at[pl.ds(rbase, CH // 128)], idx_v)
        descs = [pltpu.async_copy(h_hbm.at[idx_v.at[j]],
                                  rows_v.at[pl.ds(j * 128, 128)], sem)
                 for j in range(CH // 128)]
        for d in descs:
            d.wait()
        pltpu.sync_copy(rows_v, out_hbm.at[pl.ds(ebase, CH)])
        return ()

    lax.fori_loop(0, NCH, chunk, ())


def _scatter_body(msg_hbm, dst_hbm, zrows_hbm, out_hbm, idx_v, msg_v, agg_sh, sem):
    cid = lax.axis_index("c")
    sid = lax.axis_index("s")
    pltpu.sync_copy(zrows_hbm, agg_sh.at[pl.ds(sid * ROWS_T, ROWS_T)])
    plsc.subcore_barrier()

    def chunk(ch, _):
        ebase = cid * (EP // 2) + sid * EW + ch * CH
        rbase = ebase // 128
        pltpu.sync_copy(dst_hbm.at[pl.ds(rbase, CH // 128)], idx_v)
        pltpu.sync_copy(msg_hbm.at[pl.ds(ebase, CH)], msg_v)
        for j in range(CH // 128):
            pltpu.sync_copy(msg_v.at[pl.ds(j * 128, 128)],
                            agg_sh.at[idx_v.at[j]], add=True)
        return ()

    lax.fori_loop(0, NCH, chunk, ())
    plsc.subcore_barrier()
    pltpu.sync_copy(agg_sh.at[pl.ds(sid * ROWS_T, ROWS_T)],
                    out_hbm.at[pl.ds(cid * NP + sid * ROWS_T, ROWS_T)])


@functools.lru_cache(maxsize=1)
def _sc_mesh():
    return plsc.VectorSubcoreMesh(core_axis_name="c", subcore_axis_name="s",
                                  num_cores=2, num_subcores=16)


def _sc_gather(h, src2d):
    f = pl.kernel(
        _gather_body,
        out_type=jax.ShapeDtypeStruct((EP, D), jnp.float32),
        mesh=_sc_mesh(),
        scratch_types=[
            pltpu.VMEM((CH // 128, 128), jnp.int32),
            pltpu.VMEM((CH, D), jnp.float32),
            pltpu.SemaphoreType.DMA,
        ],
        compiler_params=pltpu.CompilerParams(use_tc_tiling_on_sc=False),
    )
    return f(h, src2d)


def _sc_scatter(msg, dst2d, zrows):
    f = pl.kernel(
        _scatter_body,
        out_type=jax.ShapeDtypeStruct((2 * NP, D), jnp.float32),
        mesh=_sc_mesh(),
        scratch_types=[
            pltpu.VMEM((CH // 128, 128), jnp.int32),
            pltpu.VMEM((CH, D), jnp.float32),
            pltpu.VMEM_SHARED((NP, D), jnp.float32),
            pltpu.SemaphoreType.DMA,
        ],
        compiler_params=pltpu.CompilerParams(use_tc_tiling_on_sc=False),
    )
    return f(msg, dst2d, zrows)


def _tc_h0(xp, w0, b0):
    return pl.pallas_call(
        _h0_body,
        out_shape=jax.ShapeDtypeStruct((N, D), jnp.float32),
    )(xp, w0, b0)


def _tc_msg(ea, hs, en1, b1, wcat, sel):
    grid = EP // BE
    return pl.pallas_call(
        _msg_body,
        grid=(grid,),
        in_specs=[
            pl.BlockSpec((BE, 8), lambda i: (i, 0)),
            pl.BlockSpec((BE, D), lambda i: (i, 0)),
            pl.BlockSpec((8, D), lambda i: (0, 0)),
            pl.BlockSpec((1, D), lambda i: (0, 0)),
            pl.BlockSpec((D, D * D + D), lambda i: (0, 0)),
            pl.BlockSpec((40, D * D + D), lambda i: (0, 0)),
        ],
        out_specs=pl.BlockSpec((BE, D), lambda i: (i, 0)),
        out_shape=jax.ShapeDtypeStruct((EP, D), jnp.float32),
    )(ea, hs, en1, b1, wcat, sel)


def _tc_update(p, h, root, cb, wih, whh, bih, bhh):
    return pl.pallas_call(
        _update_body,
        out_shape=jax.ShapeDtypeStruct((N, D), jnp.float32),
    )(p, h, root, cb, wih, whh, bih, bhh)


def _tc_s2s(h, b2d, args):
    return pl.pallas_call(
        _s2s_body,
        out_shape=(
            jax.ShapeDtypeStruct((GP, 1), jnp.float32),
            jax.ShapeDtypeStruct((GP, D), jnp.float32),
            jax.ShapeDtypeStruct((GP, D), jnp.float32),
        ),
    )(h, b2d, *args)


def kernel(x, edge_index, edge_attr, batch, lin0_W, lin0_b, en1_W, en1_b,
           en2_W, en2_b, root, conv_b, gru_Wih, gru_Whh, gru_bih, gru_bhh,
           s2s_Wih, s2s_Whh, s2s_bih, s2s_bhh, mem_Wih, mem_Whh, mem_bih,
           mem_bhh, mlp1_W, mlp1_b, mlp2_W, mlp2_b):
    f32 = jnp.float32
    src = edge_index[0]
    dst = edge_index[1]
    pad = EP - E
    src2d = jnp.concatenate([src, jnp.zeros((pad,), jnp.int32)]).reshape(EP // 128, 128)
    dst2d = jnp.concatenate([dst, jnp.full((pad,), N, jnp.int32)]).reshape(EP // 128, 128)
    ea = jnp.pad(edge_attr, ((0, pad), (0, 4)))
    xp = jnp.pad(x, ((0, 0), (0, 5)))
    w0 = jnp.pad(lin0_W, ((0, 0), (0, 5))).T
    b0 = lin0_b.reshape(1, D)
    en1 = jnp.pad(en1_W, ((0, 0), (0, 4))).T
    b1 = en1_b.reshape(1, D)
    wcat = jnp.concatenate(
        [en2_W.reshape(D, D, D).transpose(0, 2, 1).reshape(D, D * D),
         en2_b.reshape(D, D)], axis=1).astype(jnp.bfloat16)
    kk = jnp.arange(40)[:, None]
    cc = jnp.arange(D * D + D)[None, :]
    sel = (cc // D == kk).astype(jnp.bfloat16)
    wih = gru_Wih.T
    whh = gru_Whh.T
    bih = gru_bih.reshape(1, 3 * D)
    bhh = gru_bhh.reshape(1, 3 * D)
    zrows = jnp.zeros((ROWS_T, D), f32)
    b2d = batch.reshape(N, 1)

    h = _tc_h0(xp, w0, b0)
    for _ in range(6):
        hs = _sc_gather(h, src2d)
        msg = _tc_msg(ea, hs, en1, b1, wcat, sel)
        p = _sc_scatter(msg, dst2d, zrows)
        h = _tc_update(p, h, root, cb := conv_b.reshape(1, D), wih, whh, bih, bhh)

    s2s_args = (s2s_Wih.T, s2s_Whh.T, s2s_bih.reshape(1, 4 * D),
                s2s_bhh.reshape(1, 4 * D), mem_Wih.T, mem_Whh.T,
                mem_bih.reshape(1, 4 * D), mem_bhh.reshape(1, 4 * D),
                mlp1_W.T, mlp1_b.reshape(1, D), mlp2_W.T, mlp2_b.reshape(1, 1))
    v, hx, cx = _tc_s2s(h, b2d, s2s_args)
    return v[:G][None], hx[:G][None], cx[:G][None]
